# baseline (speedup 1.0000x reference)
import functools
import math

import jax
import jax.numpy as jnp
from jax import lax
from jax.experimental import pallas as pl
from jax.experimental.pallas import tpu as pltpu
from jax.experimental.pallas import tpu_sc as plsc

D_MODEL = 1024
BATCH = 4
SEQ_LEN = 2048
N_ROWS = BATCH * SEQ_LEN
SCALE = math.sqrt(D_MODEL)
LANES = 16

NUM_CORES = 2
NUM_SUBCORES = 16
NUM_WORKERS = NUM_CORES * NUM_SUBCORES
RPW = N_ROWS // NUM_WORKERS
C = 16
NCHUNKS = RPW // C


def _sc_embed(table, idx1d, pe2d):
    mesh = plsc.VectorSubcoreMesh(core_axis_name="core", subcore_axis_name="subcore")

    @functools.partial(
        pl.kernel,
        out_type=jax.ShapeDtypeStruct((N_ROWS, D_MODEL), jnp.float32),
        mesh=mesh,
        scratch_types=[
            pltpu.VMEM((RPW,), jnp.int32),
            pltpu.VMEM((C, D_MODEL), jnp.float32),
            pltpu.VMEM((C, D_MODEL), jnp.float32),
            pltpu.VMEM((C, D_MODEL), jnp.float32),
            pltpu.VMEM((C, D_MODEL), jnp.float32),
            pltpu.SemaphoreType.DMA((2,)),
            pltpu.SemaphoreType.DMA((2,)),
            pltpu.SemaphoreType.DMA((2,)),
        ],
    )
    def kern(table_hbm, idx_hbm, pe_hbm, out_hbm,
             idx_v, rows0, rows1, pe0, pe1, gsem, psem, ssem):
        wid = lax.axis_index("core") * NUM_SUBCORES + lax.axis_index("subcore")
        base = wid * RPW
        s0 = lax.rem(base, SEQ_LEN)
        pltpu.sync_copy(idx_hbm.at[pl.ds(base, RPW)], idx_v)

        rows = (rows0, rows1)
        pes = (pe0, pe1)

        def issue(k):
            b = k % 2
            g = pltpu.async_copy(
                table_hbm.at[idx_v.at[pl.ds(k * C, C)]], rows[b], gsem.at[b])
            p = pltpu.async_copy(
                pe_hbm.at[pl.ds(s0 + k * C, C)], pes[b], psem.at[b])
            return g, p

        def compute(rbuf, pbuf):
            @pl.loop(0, C)
            def _row(r):
                @pl.loop(0, D_MODEL, step=LANES)
                def _col(c):
                    sl = pl.ds(c, LANES)
                    rbuf[r, sl] = rbuf[r, sl] * SCALE + pbuf[r, sl]

        in_flight = {0: issue(0)}
        stores = {}
        for k in range(NCHUNKS):
            b = k % 2
            if k + 1 < NCHUNKS:
                if k >= 1:
                    stores[k - 1].wait()
                in_flight[k + 1] = issue(k + 1)
            g, p = in_flight.pop(k)
            g.wait()
            p.wait()
            compute(rows[b], pes[b])
            stores[k] = pltpu.async_copy(
                rows[b], out_hbm.at[pl.ds(base + k * C, C)], ssem.at[b])
        stores[NCHUNKS - 2].wait()
        stores[NCHUNKS - 1].wait()

    return kern(table, idx1d, pe2d)


def kernel(encoded_words, embed_table, pe):
    idx1d = encoded_words.astype(jnp.int32).reshape(N_ROWS)
    pe2d = pe.reshape(SEQ_LEN, D_MODEL)
    out = _sc_embed(embed_table, idx1d, pe2d)
    return out.reshape(BATCH, SEQ_LEN, D_MODEL)
